# baseline (speedup 1.0000x reference)
import math

import jax
import jax.numpy as jnp
from jax.experimental import pallas as pl
from jax.experimental.pallas import tpu as pltpu

_B, _T, _C, _H, _HS, _BS, _NB, _NS, _W = 1, 2048, 768, 12, 64, 64, 32, 8, 128
_TQ = 512
_TM = 256
_NP = _T // _TM
_NA = _T // _TQ
_SCALE = 1.0 / math.sqrt(_HS)
_NPROJ = 3 * _C + 128
_DN = (((1,), (1,)), ((), ()))
_DN0 = (((0,), (0,)), ((), ()))


def _mega_kernel(
    x_ref, w_ref, b_ref, wo_ref, bo_ref, g_ref, idx_ref,
    out_ref, r_ref, ent_ref,
    qk, vo, auxs, kc_ref, vc_ref,
):
    i = pl.program_id(0)

    @pl.when(i < _NP)
    def _proj():
        y = (
            jax.lax.dot_general(
                x_ref[...], w_ref[...], _DN, preferred_element_type=jnp.float32
            )
            + b_ref[...]
        )
        r0 = pl.multiple_of(i * _TM, _TM)
        for hh in range(_H):
            qk[hh, pl.ds(r0, _TM), 0:_HS] = y[:, hh * _HS : (hh + 1) * _HS].astype(
                jnp.bfloat16
            )
            qk[hh, pl.ds(r0, _TM), _HS:128] = y[
                :, _C + hh * _HS : _C + (hh + 1) * _HS
            ].astype(jnp.bfloat16)
            vo[hh, pl.ds(r0, _TM), 0:_HS] = y[
                :, 2 * _C + hh * _HS : 2 * _C + (hh + 1) * _HS
            ]
        auxs[pl.ds(r0, _TM), :] = y[:, 3 * _C :]
        r_ref[...] = y[:, 3 * _C : 3 * _C + _NB]

    @pl.when((i >= _NP) & (i < _NP + _H * _NA))
    def _attn():
        j = i - _NP
        h = j // _NA
        qt = j - h * _NA
        q0 = pl.multiple_of(qt * _TQ, _TQ)
        gind = g_ref[...]
        q = qk[h, pl.ds(q0, _TQ), 0:_HS]
        k = qk[h, :, _HS:128]
        v = vo[h, :, 0:_HS]

        @pl.when(qt == 0)
        def _():
            kc_ref[...] = (
                jax.lax.dot_general(
                    gind.astype(jnp.bfloat16), k, _DN0,
                    preferred_element_type=jnp.float32,
                )
                * (1.0 / _BS)
            ).astype(jnp.bfloat16)
            vc_ref[...] = jax.lax.dot_general(
                gind, v, _DN0, preferred_element_type=jnp.float32
            ) * (1.0 / _BS)

        _CK = 512
        z = jnp.zeros((_TQ, _NB), jnp.float32)
        u = jnp.zeros((_TQ, _NB), jnp.float32)
        for c in range(_T // _CK):
            kch = qk[h, c * _CK : (c + 1) * _CK, _HS:128]
            sc = (
                jax.lax.dot_general(q, kch, _DN, preferred_element_type=jnp.float32)
                * _SCALE
            )
            ec = jnp.exp(sc)
            esc = ec * sc
            gch = gind[c * _CK : (c + 1) * _CK, :]
            z = z + jnp.dot(ec, gch, preferred_element_type=jnp.float32)
            u = u + jnp.dot(esc, gch, preferred_element_type=jnp.float32)
        ent_ref[0] = jnp.log(z) - u / z

        _BAND = _TQ + _W
        koff = pl.multiple_of(jnp.maximum(q0 - _W, 0), _W)
        k_band = qk[h, pl.ds(koff, _BAND), _HS:128]
        v_band = vo[h, pl.ds(koff, _BAND), 0:_HS]
        sb = (
            jax.lax.dot_general(q, k_band, _DN, preferred_element_type=jnp.float32)
            * _SCALE
        )
        rowb = jax.lax.broadcasted_iota(jnp.int32, (_TQ, _BAND), 0) + q0
        colb = jax.lax.broadcasted_iota(jnp.int32, (_TQ, _BAND), 1) + koff
        eb = jnp.where((colb <= rowb) & (colb >= rowb - _W), jnp.exp(sb), 0.0)
        zb = jnp.sum(eb, axis=-1, keepdims=True)
        attn_sl = jnp.dot(eb, v_band, preferred_element_type=jnp.float32) / zb

        sel_k = jnp.concatenate(
            [
                qk[h, pl.ds(pl.multiple_of(idx_ref[h, sl] * _BS, _BS), _BS), _HS:128]
                for sl in range(_NS)
            ],
            axis=0,
        )
        sel_v = jnp.concatenate(
            [
                vo[h, pl.ds(pl.multiple_of(idx_ref[h, sl] * _BS, _BS), _BS), 0:_HS]
                for sl in range(_NS)
            ],
            axis=0,
        )
        ss = (
            jax.lax.dot_general(q, sel_k, _DN, preferred_element_type=jnp.float32)
            * _SCALE
        )
        rowc = jax.lax.broadcasted_iota(jnp.int32, (_TQ, _NS * _BS), 0) + q0
        cpos = jax.lax.broadcasted_iota(jnp.int32, (_TQ, _NS * _BS), 1)
        es2 = jnp.where(cpos <= rowc, jnp.exp(ss), 0.0)
        zs = jnp.sum(es2, axis=-1, keepdims=True)
        attn_sel = jnp.dot(es2, sel_v, preferred_element_type=jnp.float32) / zs

        kc = kc_ref[...]
        vc = vc_ref[...]
        cs = jax.lax.dot_general(q, kc, _DN, preferred_element_type=jnp.float32) * _SCALE
        rown = jax.lax.broadcasted_iota(jnp.int32, (_TQ, _NB), 0) + q0
        coln = jax.lax.broadcasted_iota(jnp.int32, (_TQ, _NB), 1)
        ec = jnp.where(coln <= rown, jnp.exp(cs), 0.0)
        zc = jnp.sum(ec, axis=-1, keepdims=True)
        attn_cmp = jnp.dot(ec, vc, preferred_element_type=jnp.float32) / zc

        gl = auxs[pl.ds(q0, _TQ), :][:, 32:35]
        gm = jnp.max(gl, axis=-1, keepdims=True)
        ge = jnp.exp(gl - gm)
        g = ge / jnp.sum(ge, axis=-1, keepdims=True)
        vo[h, pl.ds(q0, _TQ), _HS:128] = (
            g[:, 0:1] * attn_sl + g[:, 1:2] * attn_sel + g[:, 2:3] * attn_cmp
        )

    @pl.when(i >= _NP + _H * _NA)
    def _out():
        t = i - _NP - _H * _NA
        r0 = pl.multiple_of(t * _TM, _TM)
        acc = None
        for hh in range(_H):
            p = jax.lax.dot_general(
                vo[hh, pl.ds(r0, _TM), _HS:128],
                wo_ref[:, hh * _HS : (hh + 1) * _HS],
                _DN,
                preferred_element_type=jnp.float32,
            )
            acc = p if acc is None else acc + p
        out_ref[...] = acc + bo_ref[...]


def kernel(hidden_states, top_indices, Wq, bq, Wk, bk, Wv, bv, Wo, bo, Wg, bg, Wr, br):
    x = hidden_states.reshape(_T, _C)
    pad = _NPROJ - 3 * _C - _NB - 3
    w_stack = jnp.concatenate(
        [Wq, Wk, Wv, Wr, Wg, jnp.zeros((pad, _C), jnp.float32)], axis=0
    )
    b_all = jnp.concatenate(
        [bq, bk, bv, br, bg, jnp.zeros((pad,), jnp.float32)]
    ).reshape(1, _NPROJ)
    idx = top_indices.reshape(_H, _NS).astype(jnp.int32)
    gind = (
        jnp.arange(_T, dtype=jnp.int32)[:, None] // _BS
        == jnp.arange(_NB, dtype=jnp.int32)[None, :]
    ).astype(jnp.float32)

    def _proj_tile(i):
        return jnp.minimum(i, _NP - 1)

    def _attn_block(i):
        a = jnp.clip(i - _NP, 0, _H * _NA - 1)
        return a // _NA, a - (a // _NA) * _NA

    out, router, ent = pl.pallas_call(
        _mega_kernel,
        grid=(_NP + _H * _NA + _NP,),
        in_specs=[
            pl.BlockSpec((_TM, _C), lambda i: (_proj_tile(i), 0)),
            pl.BlockSpec((_NPROJ, _C), lambda i: (0, 0)),
            pl.BlockSpec((1, _NPROJ), lambda i: (0, 0)),
            pl.BlockSpec((_C, _C), lambda i: (0, 0)),
            pl.BlockSpec((1, _C), lambda i: (0, 0)),
            pl.BlockSpec((_T, _NB), lambda i: (0, 0)),
            pl.BlockSpec(memory_space=pltpu.SMEM),
        ],
        out_specs=[
            pl.BlockSpec(
                (_TM, _C), lambda i: (jnp.clip(i - _NP - _H * _NA, 0, _NP - 1), 0)
            ),
            pl.BlockSpec((_TM, _NB), lambda i: (_proj_tile(i), 0)),
            pl.BlockSpec(
                (1, _TQ, _NB), lambda i: (*_attn_block(i), 0)
            ),
        ],
        out_shape=[
            jax.ShapeDtypeStruct((_T, _C), jnp.float32),
            jax.ShapeDtypeStruct((_T, _NB), jnp.float32),
            jax.ShapeDtypeStruct((_H, _T, _NB), jnp.float32),
        ],
        scratch_shapes=[
            pltpu.VMEM((_H, _T, 128), jnp.bfloat16),
            pltpu.VMEM((_H, _T, 128), jnp.float32),
            pltpu.VMEM((_T, 128), jnp.float32),
            pltpu.VMEM((_NB, _HS), jnp.bfloat16),
            pltpu.VMEM((_NB, _HS), jnp.float32),
        ],
    )(x, w_stack, b_all, Wo, bo.reshape(1, _C), gind, idx)

    return out.reshape(_B, _T, _C), router.reshape(_B, _T, _NB), ent[None]


# speedup vs baseline: 1.0330x; 1.0330x over previous
import math

import jax
import jax.numpy as jnp
from jax.experimental import pallas as pl
from jax.experimental.pallas import tpu as pltpu

_B, _T, _C, _H, _HS, _BS, _NB, _NS, _W = 1, 2048, 768, 12, 64, 64, 32, 8, 128
_TQ = 512
_TM = 512
_SCALE = 1.0 / math.sqrt(_HS)
_NPROJ = 3 * _C + 128
_DN = (((1,), (1,)), ((), ()))
_DN0 = (((0,), (0,)), ((), ()))


def _proj_kernel(x_ref, w_ref, b_ref, q_ref, k_ref, v_ref, aux_ref, r_ref):
    y = (
        jax.lax.dot_general(
            x_ref[...], w_ref[...], _DN, preferred_element_type=jnp.float32
        )
        + b_ref[...]
    )
    for hh in range(_H):
        q_ref[hh] = y[:, hh * _HS : (hh + 1) * _HS].astype(jnp.bfloat16)
        k_ref[hh] = y[:, _C + hh * _HS : _C + (hh + 1) * _HS].astype(jnp.bfloat16)
        v_ref[hh] = y[:, 2 * _C + hh * _HS : 2 * _C + (hh + 1) * _HS]
    aux_ref[...] = y[:, 3 * _C :]
    r_ref[...] = y[:, 3 * _C : 3 * _C + _NB]


def _attn_kernel(q_ref, k_ref, v_ref, aux_ref, g_ref, idx_ref, o_ref, ent_ref, kc_ref, vc_ref):
    h = pl.program_id(0)
    qt = pl.program_id(1)
    q = q_ref[0]
    k = k_ref[0]
    v = v_ref[0]
    gind = g_ref[...]
    s = jax.lax.dot_general(q, k, _DN, preferred_element_type=jnp.float32) * _SCALE
    q0 = qt * _TQ

    @pl.when(qt == 0)
    def _():
        kc_ref[...] = (
            jax.lax.dot_general(
                gind.astype(jnp.bfloat16), k, _DN0, preferred_element_type=jnp.float32
            )
            * (1.0 / _BS)
        ).astype(jnp.bfloat16)
        vc_ref[...] = jax.lax.dot_general(
            gind, v, _DN0, preferred_element_type=jnp.float32
        ) * (1.0 / _BS)

    e = jnp.exp(s)
    es = e * s
    z = jnp.dot(e, gind, preferred_element_type=jnp.float32)
    u = jnp.dot(es, gind, preferred_element_type=jnp.float32)
    ent_ref[0] = jnp.log(z) - u / z

    _BAND = _TQ + _W
    koff = pl.multiple_of(jnp.maximum(q0 - _W, 0), _W)
    k_band = k_ref[0, pl.ds(koff, _BAND), :]
    v_band = v_ref[0, pl.ds(koff, _BAND), :]
    sb = jax.lax.dot_general(q, k_band, _DN, preferred_element_type=jnp.float32) * _SCALE
    rowb = jax.lax.broadcasted_iota(jnp.int32, (_TQ, _BAND), 0) + q0
    colb = jax.lax.broadcasted_iota(jnp.int32, (_TQ, _BAND), 1) + koff
    eb = jnp.where((colb <= rowb) & (colb >= rowb - _W), jnp.exp(sb), 0.0)
    zb = jnp.sum(eb, axis=-1, keepdims=True)
    attn_sl = jnp.dot(eb, v_band, preferred_element_type=jnp.float32) / zb

    sel_k = jnp.concatenate(
        [k_ref[0, pl.ds(pl.multiple_of(idx_ref[h, sl] * _BS, _BS), _BS), :] for sl in range(_NS)], axis=0
    )
    sel_v = jnp.concatenate(
        [v_ref[0, pl.ds(pl.multiple_of(idx_ref[h, sl] * _BS, _BS), _BS), :] for sl in range(_NS)], axis=0
    )
    ss = jax.lax.dot_general(q, sel_k, _DN, preferred_element_type=jnp.float32) * _SCALE
    rowc = jax.lax.broadcasted_iota(jnp.int32, (_TQ, _NS * _BS), 0) + q0
    cpos = jax.lax.broadcasted_iota(jnp.int32, (_TQ, _NS * _BS), 1)
    es2 = jnp.where(cpos <= rowc, jnp.exp(ss), 0.0)
    zs = jnp.sum(es2, axis=-1, keepdims=True)
    attn_sel = jnp.dot(es2, sel_v, preferred_element_type=jnp.float32) / zs

    kc = kc_ref[...]
    vc = vc_ref[...]
    cs = jax.lax.dot_general(q, kc, _DN, preferred_element_type=jnp.float32) * _SCALE
    rown = jax.lax.broadcasted_iota(jnp.int32, (_TQ, _NB), 0) + q0
    coln = jax.lax.broadcasted_iota(jnp.int32, (_TQ, _NB), 1)
    ec = jnp.where(coln <= rown, jnp.exp(cs), 0.0)
    zc = jnp.sum(ec, axis=-1, keepdims=True)
    attn_cmp = jnp.dot(ec, vc, preferred_element_type=jnp.float32) / zc

    gl = aux_ref[...][:, 32:35]
    gm = jnp.max(gl, axis=-1, keepdims=True)
    ge = jnp.exp(gl - gm)
    g = ge / jnp.sum(ge, axis=-1, keepdims=True)
    o_ref[0] = (
        g[:, 0:1] * attn_sl + g[:, 1:2] * attn_sel + g[:, 2:3] * attn_cmp
    )


def _out_kernel(o_ref, w_ref, b_ref, y_ref):
    acc = None
    for hh in range(_H):
        p = jax.lax.dot_general(
            o_ref[hh],
            w_ref[:, hh * _HS : (hh + 1) * _HS],
            _DN,
            preferred_element_type=jnp.float32,
        )
        acc = p if acc is None else acc + p
    y_ref[...] = acc + b_ref[...]


def kernel(hidden_states, top_indices, Wq, bq, Wk, bk, Wv, bv, Wo, bo, Wg, bg, Wr, br):
    x = hidden_states.reshape(_T, _C)
    pad = _NPROJ - 3 * _C - _NB - 3
    w_stack = jnp.concatenate(
        [Wq, Wk, Wv, Wr, Wg, jnp.zeros((pad, _C), jnp.float32)], axis=0
    )
    b_all = jnp.concatenate(
        [bq, bk, bv, br, bg, jnp.zeros((pad,), jnp.float32)]
    ).reshape(1, _NPROJ)

    q3, k3, v3, aux, router = pl.pallas_call(
        _proj_kernel,
        grid=(_T // _TM,),
        in_specs=[
            pl.BlockSpec((_TM, _C), lambda i: (i, 0)),
            pl.BlockSpec((_NPROJ, _C), lambda i: (0, 0)),
            pl.BlockSpec((1, _NPROJ), lambda i: (0, 0)),
        ],
        out_specs=[
            pl.BlockSpec((_H, _TM, _HS), lambda i: (0, i, 0)),
            pl.BlockSpec((_H, _TM, _HS), lambda i: (0, i, 0)),
            pl.BlockSpec((_H, _TM, _HS), lambda i: (0, i, 0)),
            pl.BlockSpec((_TM, 128), lambda i: (i, 0)),
            pl.BlockSpec((_TM, _NB), lambda i: (i, 0)),
        ],
        out_shape=[
            jax.ShapeDtypeStruct((_H, _T, _HS), jnp.bfloat16),
            jax.ShapeDtypeStruct((_H, _T, _HS), jnp.bfloat16),
            jax.ShapeDtypeStruct((_H, _T, _HS), jnp.float32),
            jax.ShapeDtypeStruct((_T, 128), jnp.float32),
            jax.ShapeDtypeStruct((_T, _NB), jnp.float32),
        ],
    )(x, w_stack, b_all)

    router_logits = router.reshape(_B, _T, _NB)
    idx = top_indices.reshape(_H, _NS).astype(jnp.int32)
    gind = (
        jnp.arange(_T, dtype=jnp.int32)[:, None] // _BS
        == jnp.arange(_NB, dtype=jnp.int32)[None, :]
    ).astype(jnp.float32)

    o3, ent = pl.pallas_call(
        _attn_kernel,
        grid=(_H, _T // _TQ),
        in_specs=[
            pl.BlockSpec((1, _TQ, _HS), lambda h, i: (h, i, 0)),
            pl.BlockSpec((1, _T, _HS), lambda h, i: (h, 0, 0)),
            pl.BlockSpec((1, _T, _HS), lambda h, i: (h, 0, 0)),
            pl.BlockSpec((_TQ, 128), lambda h, i: (i, 0)),
            pl.BlockSpec((_T, _NB), lambda h, i: (0, 0)),
            pl.BlockSpec(memory_space=pltpu.SMEM),
        ],
        out_specs=[
            pl.BlockSpec((1, _TQ, _HS), lambda h, i: (h, i, 0)),
            pl.BlockSpec((1, _TQ, _NB), lambda h, i: (h, i, 0)),
        ],
        out_shape=[
            jax.ShapeDtypeStruct((_H, _T, _HS), jnp.float32),
            jax.ShapeDtypeStruct((_H, _T, _NB), jnp.float32),
        ],
        scratch_shapes=[
            pltpu.VMEM((_NB, _HS), jnp.bfloat16),
            pltpu.VMEM((_NB, _HS), jnp.float32),
        ],
    )(q3, k3, v3, aux, gind, idx)

    out = pl.pallas_call(
        _out_kernel,
        grid=(_T // _TM,),
        in_specs=[
            pl.BlockSpec((_H, _TM, _HS), lambda i: (0, i, 0)),
            pl.BlockSpec((_C, _C), lambda i: (0, 0)),
            pl.BlockSpec((1, _C), lambda i: (0, 0)),
        ],
        out_specs=pl.BlockSpec((_TM, _C), lambda i: (i, 0)),
        out_shape=jax.ShapeDtypeStruct((_T, _C), jnp.float32),
    )(o3, Wo, bo.reshape(1, _C))

    return out.reshape(_B, _T, _C), router_logits, ent[None]
